# Initial kernel scaffold; baseline (speedup 1.0000x reference)
import jax
import jax.numpy as jnp
from jax.experimental import pallas as pl


def kernel(x_src, loc_src, loc_tar):
    raise NotImplementedError("write your pallas kernel here")


# speedup vs baseline: 34.1296x; 34.1296x over previous
import functools

import jax
import jax.numpy as jnp
from jax.experimental import pallas as pl


TILE = 256


def _resample_tile(loc_tar_ref, loc_src_ref, x_src_ref, out_ref):
    lt = loc_tar_ref[0]
    ls = loc_src_ref[0]
    xs = x_src_ref[0]

    n_src = ls.shape[0]

    tx = lt[:, 0:1]
    ty = lt[:, 1:2]
    sx = ls[:, 0].reshape(1, n_src)
    sy = ls[:, 1].reshape(1, n_src)

    dx = tx - sx
    dy = ty - sy
    dist = dx * dx + dy * dy

    src_iota = jax.lax.broadcasted_iota(jnp.int32, dist.shape, 1)

    d_list = []
    idx_list = []
    d = dist
    for _ in range(3):
        a = jnp.argmin(d, axis=1).astype(jnp.int32)
        m = jnp.min(d, axis=1)
        d_list.append(m)
        idx_list.append(a)
        d = jnp.where(src_iota == a[:, None], jnp.inf, d)

    d3 = jnp.stack(d_list, axis=1)
    dist_recip = 1.0 / (d3 + 1e-06)
    one_mask = d3 == 0.0
    zero_mask = jnp.sum(one_mask, axis=-1) > 0
    dist_recip = jnp.where(zero_mask[:, None], 0.0, dist_recip)
    dist_recip = jnp.where(one_mask, 1.0, dist_recip)
    norm = jnp.sum(dist_recip, axis=1, keepdims=True)
    weight = dist_recip / norm

    w_mat = jnp.zeros(dist.shape, dtype=jnp.float32)
    for k in range(3):
        w_mat = w_mat + jnp.where(
            src_iota == idx_list[k][:, None], weight[:, k][:, None], 0.0
        )

    out_ref[0] = jnp.dot(w_mat, xs, preferred_element_type=jnp.float32)


@functools.partial(jax.jit, static_argnames=())
def kernel(x_src, loc_src, loc_tar):
    B, N_src, C = x_src.shape
    _, N_tar, _ = loc_tar.shape

    grid = (B, N_tar // TILE)
    return pl.pallas_call(
        _resample_tile,
        grid=grid,
        in_specs=[
            pl.BlockSpec((1, TILE, 2), lambda b, t: (b, t, 0)),
            pl.BlockSpec((1, N_src, 2), lambda b, t: (b, 0, 0)),
            pl.BlockSpec((1, N_src, C), lambda b, t: (b, 0, 0)),
        ],
        out_specs=pl.BlockSpec((1, TILE, C), lambda b, t: (b, t, 0)),
        out_shape=jax.ShapeDtypeStruct((B, N_tar, C), jnp.float32),
    )(loc_tar, loc_src, x_src)
